# baseline (speedup 1.0000x reference)
import functools

import jax
import jax.numpy as jnp
from jax import lax
from jax.experimental import pallas as pl
from jax.experimental.pallas import tpu as pltpu
from jax.experimental.pallas import tpu_sc as plsc

NN = 10000
EE = 320000
DD = 128
RR = 10
KEYS = RR * NN
KEYS_PAD = 100096
CHUNK = 128
NTILES = 32
EP = 323584
EPW = EP // NTILES
NCHUNK = EPW // CHUNK
N_ACC = 10112
ROWS_PER_TILE_H = KEYS_PAD // 16
ROWS_PER_TILE_A = N_ACC // 16

_mesh = plsc.VectorSubcoreMesh(core_axis_name="c", subcore_axis_name="s")


HK = 896
HROWS_PER_TILE = HK // 16


def _sc_hist(dst_p, typ_p, iota7):
    @functools.partial(
        pl.kernel,
        out_type=jax.ShapeDtypeStruct((2, HK, 128), jnp.float32),
        mesh=_mesh,
        compiler_params=pltpu.CompilerParams(needs_layout_passes=False),
        scratch_types=[
            pltpu.VMEM((CHUNK + 16,), jnp.int32),
            pltpu.VMEM((CHUNK + 16,), jnp.int32),
            pltpu.VMEM((7, 128), jnp.int32),
            pltpu.VMEM((HK, 128), jnp.float32),
            pltpu.VMEM_SHARED((HK, 128), jnp.float32),
        ],
    )
    def hist_kernel(dst_hbm, typ_hbm, iota_hbm, out_hbm, dst_v, typ_v,
                    idx_v, cnt_v, hist_sh):
        c = lax.axis_index("c")
        s = lax.axis_index("s")
        wid = c * 16 + s

        pltpu.sync_copy(iota_hbm, idx_v)

        def zrow(i, _):
            for j in range(128 // 16):
                cnt_v[i, pl.ds(j * 16, 16)] = jnp.zeros((16,), jnp.float32)
            return 0
        lax.fori_loop(0, HK, zrow, 0)

        pltpu.sync_copy(cnt_v.at[pl.ds(0, HROWS_PER_TILE), :],
                        hist_sh.at[pl.ds(s * HROWS_PER_TILE, HROWS_PER_TILE), :])
        plsc.subcore_barrier()

        lane0 = lax.iota(jnp.int32, 16) == 0

        def edge_body(g, _):
            base = wid * EPW + g * CHUNK
            pltpu.sync_copy(dst_hbm.at[pl.ds(base, CHUNK)],
                            dst_v.at[pl.ds(0, CHUNK)])
            pltpu.sync_copy(typ_hbm.at[pl.ds(base, CHUNK)],
                            typ_v.at[pl.ds(0, CHUNK)])

            def one_edge(e, _):
                k = typ_v[pl.ds(e, 16)][0] * NN + dst_v[pl.ds(e, 16)][0]
                r16 = jnp.full((16,), lax.shift_right_logical(k, 7), jnp.int32)
                l16 = jnp.full((16,), lax.bitwise_and(k, 127), jnp.int32)
                cur = plsc.load_gather(cnt_v, [r16, l16])
                plsc.store_scatter(cnt_v, [r16, l16], cur + 1.0, mask=lane0)
                return 0
            lax.fori_loop(0, CHUNK, one_edge, 0)
            return 0
        lax.fori_loop(0, NCHUNK, edge_body, 0)

        for j in range(7):
            pltpu.sync_copy(cnt_v.at[pl.ds(j * 128, 128), :],
                            hist_sh.at[idx_v.at[j]], add=True)
        plsc.subcore_barrier()

        pltpu.sync_copy(hist_sh.at[pl.ds(s * HROWS_PER_TILE, HROWS_PER_TILE), :],
                        out_hbm.at[c, pl.ds(s * HROWS_PER_TILE, HROWS_PER_TILE), :])

    return hist_kernel(dst_p, typ_p, iota7)


CH_A = 64
NCHUNK_A = EPW // CH_A
EPC = EP // CH_A


def _sc_aggregate(pk, t_tab, w_tab):
    @functools.partial(
        pl.kernel,
        out_type=jax.ShapeDtypeStruct((2, N_ACC, DD), jnp.float32),
        mesh=_mesh,
        compiler_params=pltpu.CompilerParams(needs_layout_passes=False),
        scratch_types=[
            pltpu.VMEM((3, CH_A), jnp.int32),
            pltpu.VMEM((3, CH_A), jnp.int32),
            pltpu.VMEM((CH_A, DD), jnp.float32),
            pltpu.VMEM((CH_A, DD), jnp.float32),
            pltpu.VMEM((CH_A, DD), jnp.float32),
            pltpu.VMEM((CH_A, DD), jnp.float32),
            pltpu.SemaphoreType.DMA,
            pltpu.SemaphoreType.DMA,
            pltpu.SemaphoreType.DMA,
            pltpu.SemaphoreType.DMA,
            pltpu.VMEM_SHARED((N_ACC, DD), jnp.float32),
        ],
    )
    def agg_kernel(pk_hbm, ttab_hbm, wtab_hbm, out_hbm,
                   ebuf0, ebuf1, rows0, rows1, wrows0, wrows1, sem0, sem1,
                   ssem0, ssem1, acc_sh):
        c = lax.axis_index("c")
        s = lax.axis_index("s")
        wid = c * 16 + s
        ebufs = (ebuf0, ebuf1)
        rows = (rows0, rows1)
        wrows = (wrows0, wrows1)
        sems = (sem0, sem1)
        ssems = (ssem0, ssem1)

        def zinit_body(i, _):
            for j in range(DD // 16):
                rows0[i, pl.ds(j * 16, 16)] = jnp.zeros((16,), jnp.float32)
            return 0
        lax.fori_loop(0, CH_A, zinit_body, 0)

        def zero_body(i, _):
            pltpu.sync_copy(
                rows0, acc_sh.at[pl.ds(s * ROWS_PER_TILE_A + i * CH_A, CH_A), :])
            return 0
        lax.fori_loop(0, 9, zero_body, 0)
        pltpu.sync_copy(rows0.at[pl.ds(0, 56)],
                        acc_sh.at[pl.ds(s * ROWS_PER_TILE_A + 9 * CH_A, 56), :])
        plsc.subcore_barrier()

        def stage_fire(b, g):
            pltpu.sync_copy(pk_hbm.at[wid * NCHUNK_A + g], ebufs[b])
            pltpu.async_copy(ttab_hbm.at[ebufs[b].at[0]], rows[b], sems[b])
            pltpu.async_copy(wtab_hbm.at[ebufs[b].at[1]], wrows[b], sems[b])

        def wait_scale_fire(b):
            pltpu.make_async_copy(ttab_hbm.at[ebufs[b].at[0]], rows[b],
                                  sems[b]).wait()
            pltpu.make_async_copy(wtab_hbm.at[ebufs[b].at[1]], wrows[b],
                                  sems[b]).wait()

            @plsc.parallel_loop(0, CH_A, unroll=2)
            def scale_body(e):
                for j in range(DD // 16):
                    sl = pl.ds(j * 16, 16)
                    rows[b][e, sl] = rows[b][e, sl] * wrows[b][e, sl]

            pltpu.async_copy(rows[b], acc_sh.at[ebufs[b].at[2]], ssems[b],
                             add=True)

        def wait_scatter(b):
            pltpu.make_async_copy(rows[b], acc_sh.at[ebufs[b].at[2]],
                                  ssems[b]).wait()

        stage_fire(0, 0)

        def edge_body(i, _):
            @pl.when(i > 0)
            def _():
                wait_scatter(1)
            stage_fire(1, 2 * i + 1)
            wait_scale_fire(0)
            wait_scale_fire(1)

            @pl.when(i < NCHUNK_A // 2 - 1)
            def _():
                wait_scatter(0)
                stage_fire(0, 2 * i + 2)
            return 0
        lax.fori_loop(0, NCHUNK_A // 2, edge_body, 0)
        wait_scatter(0)
        wait_scatter(1)
        plsc.subcore_barrier()

        plsc.subcore_barrier()

        pltpu.sync_copy(acc_sh.at[pl.ds(s * ROWS_PER_TILE_A, ROWS_PER_TILE_A), :],
                        out_hbm.at[c, pl.ds(s * ROWS_PER_TILE_A, ROWS_PER_TILE_A), :])

    return agg_kernel(pk, t_tab, w_tab)


def _tc_pack_keys(src_p, dst_p, typ_p):
    s2 = src_p.reshape(EPC, CH_A)
    d2 = dst_p.reshape(EPC, CH_A)
    t2 = typ_p.reshape(EPC, CH_A)

    def body(s_ref, d_ref, t_ref, kt_ref, kw_ref, kd_ref):
        t = t_ref[:] * NN
        kt_ref[:] = t + s_ref[:]
        kw_ref[:] = t + d_ref[:]
        kd_ref[:] = d_ref[:]

    kt, kw, kd = pl.pallas_call(
        body,
        grid=(1,),
        in_specs=[pl.BlockSpec((EPC, CH_A), lambda i: (0, 0))] * 3,
        out_specs=[pl.BlockSpec((EPC, CH_A), lambda i: (0, 0))] * 3,
        out_shape=[jax.ShapeDtypeStruct((EPC, CH_A), jnp.int32)] * 3,
    )(s2, d2, t2)
    return jnp.stack([kt, kw, kd], axis=1)


_BN = 400


def _tc_rel_transform(h, wrel):
    def body(h_ref, w_ref, o_ref):
        o_ref[:] = jnp.dot(h_ref[:], w_ref[0],
                           preferred_element_type=jnp.float32)

    nb = NN // _BN
    return pl.pallas_call(
        body,
        grid=(RR, nb),
        in_specs=[
            pl.BlockSpec((_BN, DD), lambda r, n: (n, 0)),
            pl.BlockSpec((1, DD, DD), lambda r, n: (r, 0, 0)),
        ],
        out_specs=pl.BlockSpec((_BN, DD), lambda r, n: (r * nb + n, 0)),
        out_shape=jax.ShapeDtypeStruct((KEYS, DD), jnp.float32),
    )(h, wrel)


def _tc_weight_table(hist2):
    def body(a_ref, b_ref, o_ref):
        cnt = a_ref[0] + b_ref[0]
        o_ref[:] = 1.0 / jnp.maximum(cnt, 1.0)

    return pl.pallas_call(
        body,
        grid=(1,),
        in_specs=[
            pl.BlockSpec((1, HK, 128), lambda i: (0, 0, 0)),
            pl.BlockSpec((1, HK, 128), lambda i: (1, 0, 0)),
        ],
        out_specs=pl.BlockSpec((HK, 128), lambda i: (0, 0)),
        out_shape=jax.ShapeDtypeStruct((HK, 128), jnp.float32),
    )(hist2, hist2)


def _tc_combine(h, wroot, bias, gamma, beta, agg, relu, residual):
    b2 = bias.reshape(1, DD)
    g2 = gamma.reshape(1, DD)
    be2 = beta.reshape(1, DD)

    def body(h_ref, w_ref, b_ref, g_ref, be_ref, a0_ref, a1_ref, o_ref):
        y = jnp.dot(h_ref[:], w_ref[:], preferred_element_type=jnp.float32)
        y = y + b_ref[:] + a0_ref[0] + a1_ref[0]
        mu = jnp.mean(y, axis=-1, keepdims=True)
        yc = y - mu
        var = jnp.mean(yc * yc, axis=-1, keepdims=True)
        y = yc * lax.rsqrt(var + 1e-5) * g_ref[:] + be_ref[:]
        if relu:
            y = jnp.maximum(y, 0.0)
        if residual:
            y = y + h_ref[:]
        o_ref[:] = y

    return pl.pallas_call(
        body,
        grid=(NN // _BN,),
        in_specs=[
            pl.BlockSpec((_BN, DD), lambda n: (n, 0)),
            pl.BlockSpec((DD, DD), lambda n: (0, 0)),
            pl.BlockSpec((1, DD), lambda n: (0, 0)),
            pl.BlockSpec((1, DD), lambda n: (0, 0)),
            pl.BlockSpec((1, DD), lambda n: (0, 0)),
            pl.BlockSpec((1, _BN, DD), lambda n: (0, n, 0)),
            pl.BlockSpec((1, _BN, DD), lambda n: (1, n, 0)),
        ],
        out_specs=pl.BlockSpec((_BN, DD), lambda n: (n, 0)),
        out_shape=jax.ShapeDtypeStruct((NN, DD), jnp.float32),
    )(h, wroot, b2, g2, be2, agg, agg)


def kernel(x, edge_index, edge_type, Wrel0, Wroot0, b0, g0, be0,
           Wrel1, Wroot1, b1, g1, be1):
    src = edge_index[0]
    dst = edge_index[1]
    pad = EP - EE
    src_p = jnp.concatenate([src, jnp.zeros((pad,), jnp.int32)])
    dst_p = jnp.concatenate([dst, jnp.full((pad,), NN, jnp.int32)])
    typ_p = jnp.concatenate([edge_type, jnp.full((pad,), RR - 1, jnp.int32)])

    iota7 = jnp.arange(HK, dtype=jnp.int32).reshape(7, 128)
    hist2 = _sc_hist(dst_p, typ_p, iota7)
    w_inv = _tc_weight_table(hist2)
    w_tab = jnp.broadcast_to(w_inv.reshape(HK * 128, 1), (HK * 128, DD))

    pk = _tc_pack_keys(src_p, dst_p, typ_p)

    t0 = _tc_rel_transform(x, Wrel0)
    agg0 = _sc_aggregate(pk, t0, w_tab)
    h1 = _tc_combine(x, Wroot0, b0, g0, be0, agg0, relu=True, residual=False)

    t1 = _tc_rel_transform(h1, Wrel1)
    agg1 = _sc_aggregate(pk, t1, w_tab)
    h2 = _tc_combine(h1, Wroot1, b1, g1, be1, agg1, relu=False, residual=True)
    return h2


# speedup vs baseline: 1.0886x; 1.0886x over previous
import functools

import jax
import jax.numpy as jnp
from jax import lax
from jax.experimental import pallas as pl
from jax.experimental.pallas import tpu as pltpu
from jax.experimental.pallas import tpu_sc as plsc

NN = 10000
EE = 320000
DD = 128
RR = 10
KEYS = RR * NN
KEYS_PAD = 100096
CHUNK = 128
NTILES = 32
EP = 323584
EPW = EP // NTILES
NCHUNK = EPW // CHUNK
N_ACC = 10112
ROWS_PER_TILE_H = KEYS_PAD // 16
ROWS_PER_TILE_A = N_ACC // 16

_mesh = plsc.VectorSubcoreMesh(core_axis_name="c", subcore_axis_name="s")


HK = 896
HROWS_PER_TILE = HK // 16


def _sc_hist(dst_p, typ_p, iota7):
    @functools.partial(
        pl.kernel,
        out_type=jax.ShapeDtypeStruct((2, HK, 128), jnp.float32),
        mesh=_mesh,
        compiler_params=pltpu.CompilerParams(needs_layout_passes=False),
        scratch_types=[
            pltpu.VMEM((CHUNK + 16,), jnp.int32),
            pltpu.VMEM((CHUNK + 16,), jnp.int32),
            pltpu.VMEM((7, 128), jnp.int32),
            pltpu.VMEM((HK, 128), jnp.float32),
            pltpu.VMEM_SHARED((HK, 128), jnp.float32),
        ],
    )
    def hist_kernel(dst_hbm, typ_hbm, iota_hbm, out_hbm, dst_v, typ_v,
                    idx_v, cnt_v, hist_sh):
        c = lax.axis_index("c")
        s = lax.axis_index("s")
        wid = c * 16 + s

        pltpu.sync_copy(iota_hbm, idx_v)

        def zrow(i, _):
            for j in range(128 // 16):
                cnt_v[i, pl.ds(j * 16, 16)] = jnp.zeros((16,), jnp.float32)
            return 0
        lax.fori_loop(0, HK, zrow, 0)

        pltpu.sync_copy(cnt_v.at[pl.ds(0, HROWS_PER_TILE), :],
                        hist_sh.at[pl.ds(s * HROWS_PER_TILE, HROWS_PER_TILE), :])
        plsc.subcore_barrier()

        lane0 = lax.iota(jnp.int32, 16) == 0

        def edge_body(g, _):
            base = wid * EPW + g * CHUNK
            pltpu.sync_copy(dst_hbm.at[pl.ds(base, CHUNK)],
                            dst_v.at[pl.ds(0, CHUNK)])
            pltpu.sync_copy(typ_hbm.at[pl.ds(base, CHUNK)],
                            typ_v.at[pl.ds(0, CHUNK)])

            def one_edge(e, _):
                k = typ_v[pl.ds(e, 16)][0] * NN + dst_v[pl.ds(e, 16)][0]
                r16 = jnp.full((16,), lax.shift_right_logical(k, 7), jnp.int32)
                l16 = jnp.full((16,), lax.bitwise_and(k, 127), jnp.int32)
                cur = plsc.load_gather(cnt_v, [r16, l16])
                plsc.store_scatter(cnt_v, [r16, l16], cur + 1.0, mask=lane0)
                return 0
            lax.fori_loop(0, CHUNK, one_edge, 0)
            return 0
        lax.fori_loop(0, NCHUNK, edge_body, 0)

        for j in range(7):
            pltpu.sync_copy(cnt_v.at[pl.ds(j * 128, 128), :],
                            hist_sh.at[idx_v.at[j]], add=True)
        plsc.subcore_barrier()

        pltpu.sync_copy(hist_sh.at[pl.ds(s * HROWS_PER_TILE, HROWS_PER_TILE), :],
                        out_hbm.at[c, pl.ds(s * HROWS_PER_TILE, HROWS_PER_TILE), :])

    return hist_kernel(dst_p, typ_p, iota7)


CH_A = 64
NCHUNK_A = EPW // CH_A
EPC = EP // CH_A


def _sc_aggregate(pk, t_tab, w_tab):
    @functools.partial(
        pl.kernel,
        out_type=jax.ShapeDtypeStruct((2, N_ACC, DD), jnp.float32),
        mesh=_mesh,
        compiler_params=pltpu.CompilerParams(needs_layout_passes=False),
        scratch_types=[
            pltpu.VMEM((3, CH_A), jnp.int32),
            pltpu.VMEM((3, CH_A), jnp.int32),
            pltpu.VMEM((CH_A, DD), jnp.float32),
            pltpu.VMEM((CH_A, DD), jnp.float32),
            pltpu.VMEM((CH_A, DD), jnp.float32),
            pltpu.VMEM((CH_A, DD), jnp.float32),
            pltpu.SemaphoreType.DMA,
            pltpu.SemaphoreType.DMA,
            pltpu.SemaphoreType.DMA,
            pltpu.SemaphoreType.DMA,
            pltpu.VMEM_SHARED((N_ACC, DD), jnp.float32),
        ],
    )
    def agg_kernel(pk_hbm, ttab_hbm, wtab_hbm, out_hbm,
                   ebuf0, ebuf1, rows0, rows1, wrows0, wrows1, sem0, sem1,
                   ssem0, ssem1, acc_sh):
        c = lax.axis_index("c")
        s = lax.axis_index("s")
        wid = c * 16 + s
        ebufs = (ebuf0, ebuf1)
        rows = (rows0, rows1)
        wrows = (wrows0, wrows1)
        sems = (sem0, sem1)
        ssems = (ssem0, ssem1)

        def zinit_body(i, _):
            for j in range(DD // 16):
                rows0[i, pl.ds(j * 16, 16)] = jnp.zeros((16,), jnp.float32)
            return 0
        lax.fori_loop(0, CH_A, zinit_body, 0)

        def zero_body(i, _):
            pltpu.sync_copy(
                rows0, acc_sh.at[pl.ds(s * ROWS_PER_TILE_A + i * CH_A, CH_A), :])
            return 0
        lax.fori_loop(0, 9, zero_body, 0)
        pltpu.sync_copy(rows0.at[pl.ds(0, 56)],
                        acc_sh.at[pl.ds(s * ROWS_PER_TILE_A + 9 * CH_A, 56), :])
        plsc.subcore_barrier()

        def stage_fire(b, g):
            pltpu.sync_copy(pk_hbm.at[wid * NCHUNK_A + g], ebufs[b])
            pltpu.async_copy(ttab_hbm.at[ebufs[b].at[0]], rows[b], sems[b])
            pltpu.async_copy(wtab_hbm.at[ebufs[b].at[1]], wrows[b], sems[b])

        def wait_scale_fire(b):
            pltpu.make_async_copy(ttab_hbm.at[ebufs[b].at[0]], rows[b],
                                  sems[b]).wait()
            pltpu.make_async_copy(wtab_hbm.at[ebufs[b].at[1]], wrows[b],
                                  sems[b]).wait()

            @plsc.parallel_loop(0, CH_A, unroll=2)
            def scale_body(e):
                for j in range(DD // 16):
                    sl = pl.ds(j * 16, 16)
                    rows[b][e, sl] = rows[b][e, sl] * wrows[b][e, sl]

            pltpu.sync_copy(rows[b], acc_sh.at[ebufs[b].at[2]], add=True)

        stage_fire(0, 0)

        def edge_body(i, _):
            g1 = 2 * i + 1
            g2 = 2 * i + 2
            stage_fire(1, g1)
            wait_scale_fire(0)

            @pl.when(g2 < NCHUNK_A)
            def _():
                stage_fire(0, g2)
            wait_scale_fire(1)
            return 0
        lax.fori_loop(0, NCHUNK_A // 2, edge_body, 0)
        plsc.subcore_barrier()

        plsc.subcore_barrier()

        pltpu.sync_copy(acc_sh.at[pl.ds(s * ROWS_PER_TILE_A, ROWS_PER_TILE_A), :],
                        out_hbm.at[c, pl.ds(s * ROWS_PER_TILE_A, ROWS_PER_TILE_A), :])

    return agg_kernel(pk, t_tab, w_tab)


def _tc_pack_keys(src_p, dst_p, typ_p):
    s2 = src_p.reshape(EPC, CH_A)
    d2 = dst_p.reshape(EPC, CH_A)
    t2 = typ_p.reshape(EPC, CH_A)

    def body(s_ref, d_ref, t_ref, kt_ref, kw_ref, kd_ref):
        t = t_ref[:] * NN
        kt_ref[:] = t + s_ref[:]
        kw_ref[:] = t + d_ref[:]
        kd_ref[:] = d_ref[:]

    kt, kw, kd = pl.pallas_call(
        body,
        grid=(1,),
        in_specs=[pl.BlockSpec((EPC, CH_A), lambda i: (0, 0))] * 3,
        out_specs=[pl.BlockSpec((EPC, CH_A), lambda i: (0, 0))] * 3,
        out_shape=[jax.ShapeDtypeStruct((EPC, CH_A), jnp.int32)] * 3,
    )(s2, d2, t2)
    return jnp.stack([kt, kw, kd], axis=1)


_BN = 400


def _tc_rel_transform(h, wrel):
    def body(h_ref, w_ref, o_ref):
        o_ref[:] = jnp.dot(h_ref[:], w_ref[0],
                           preferred_element_type=jnp.float32)

    nb = NN // _BN
    return pl.pallas_call(
        body,
        grid=(RR, nb),
        in_specs=[
            pl.BlockSpec((_BN, DD), lambda r, n: (n, 0)),
            pl.BlockSpec((1, DD, DD), lambda r, n: (r, 0, 0)),
        ],
        out_specs=pl.BlockSpec((_BN, DD), lambda r, n: (r * nb + n, 0)),
        out_shape=jax.ShapeDtypeStruct((KEYS, DD), jnp.float32),
    )(h, wrel)


def _tc_weight_table(hist2):
    def body(a_ref, b_ref, o_ref):
        cnt = a_ref[0] + b_ref[0]
        o_ref[:] = 1.0 / jnp.maximum(cnt, 1.0)

    return pl.pallas_call(
        body,
        grid=(1,),
        in_specs=[
            pl.BlockSpec((1, HK, 128), lambda i: (0, 0, 0)),
            pl.BlockSpec((1, HK, 128), lambda i: (1, 0, 0)),
        ],
        out_specs=pl.BlockSpec((HK, 128), lambda i: (0, 0)),
        out_shape=jax.ShapeDtypeStruct((HK, 128), jnp.float32),
    )(hist2, hist2)


def _tc_combine(h, wroot, bias, gamma, beta, agg, relu, residual):
    b2 = bias.reshape(1, DD)
    g2 = gamma.reshape(1, DD)
    be2 = beta.reshape(1, DD)

    def body(h_ref, w_ref, b_ref, g_ref, be_ref, a0_ref, a1_ref, o_ref):
        y = jnp.dot(h_ref[:], w_ref[:], preferred_element_type=jnp.float32)
        y = y + b_ref[:] + a0_ref[0] + a1_ref[0]
        mu = jnp.mean(y, axis=-1, keepdims=True)
        yc = y - mu
        var = jnp.mean(yc * yc, axis=-1, keepdims=True)
        y = yc * lax.rsqrt(var + 1e-5) * g_ref[:] + be_ref[:]
        if relu:
            y = jnp.maximum(y, 0.0)
        if residual:
            y = y + h_ref[:]
        o_ref[:] = y

    return pl.pallas_call(
        body,
        grid=(NN // _BN,),
        in_specs=[
            pl.BlockSpec((_BN, DD), lambda n: (n, 0)),
            pl.BlockSpec((DD, DD), lambda n: (0, 0)),
            pl.BlockSpec((1, DD), lambda n: (0, 0)),
            pl.BlockSpec((1, DD), lambda n: (0, 0)),
            pl.BlockSpec((1, DD), lambda n: (0, 0)),
            pl.BlockSpec((1, _BN, DD), lambda n: (0, n, 0)),
            pl.BlockSpec((1, _BN, DD), lambda n: (1, n, 0)),
        ],
        out_specs=pl.BlockSpec((_BN, DD), lambda n: (n, 0)),
        out_shape=jax.ShapeDtypeStruct((NN, DD), jnp.float32),
    )(h, wroot, b2, g2, be2, agg, agg)


def kernel(x, edge_index, edge_type, Wrel0, Wroot0, b0, g0, be0,
           Wrel1, Wroot1, b1, g1, be1):
    src = edge_index[0]
    dst = edge_index[1]
    pad = EP - EE
    src_p = jnp.concatenate([src, jnp.zeros((pad,), jnp.int32)])
    dst_p = jnp.concatenate([dst, jnp.full((pad,), NN, jnp.int32)])
    typ_p = jnp.concatenate([edge_type, jnp.full((pad,), RR - 1, jnp.int32)])

    iota7 = jnp.arange(HK, dtype=jnp.int32).reshape(7, 128)
    hist2 = _sc_hist(dst_p, typ_p, iota7)
    w_inv = _tc_weight_table(hist2)
    w_tab = jnp.broadcast_to(w_inv.reshape(HK * 128, 1), (HK * 128, DD))

    pk = _tc_pack_keys(src_p, dst_p, typ_p)

    t0 = _tc_rel_transform(x, Wrel0)
    agg0 = _sc_aggregate(pk, t0, w_tab)
    h1 = _tc_combine(x, Wroot0, b0, g0, be0, agg0, relu=True, residual=False)

    t1 = _tc_rel_transform(h1, Wrel1)
    agg1 = _sc_aggregate(pk, t1, w_tab)
    h2 = _tc_combine(h1, Wroot1, b1, g1, be1, agg1, relu=False, residual=True)
    return h2


# speedup vs baseline: 1.1732x; 1.0777x over previous
import functools

import jax
import jax.numpy as jnp
from jax import lax
from jax.experimental import pallas as pl
from jax.experimental.pallas import tpu as pltpu
from jax.experimental.pallas import tpu_sc as plsc

NN = 10000
EE = 320000
DD = 128
RR = 10
KEYS = RR * NN
KEYS_PAD = 100096
CHUNK = 128
NTILES = 32
EP = 323584
EPW = EP // NTILES
NCHUNK = EPW // CHUNK
N_ACC = 10112
ROWS_PER_TILE_H = KEYS_PAD // 16
ROWS_PER_TILE_A = N_ACC // 16

_mesh = plsc.VectorSubcoreMesh(core_axis_name="c", subcore_axis_name="s")


HK = 896
HROWS_PER_TILE = HK // 16


def _sc_hist(pk):
    @functools.partial(
        pl.kernel,
        out_type=jax.ShapeDtypeStruct((2, HK, 128), jnp.float32),
        mesh=_mesh,
        compiler_params=pltpu.CompilerParams(needs_layout_passes=False),
        scratch_types=[
            pltpu.VMEM((3, CH_A), jnp.int32),
            pltpu.VMEM((3, CH_A), jnp.int32),
            pltpu.VMEM((CH_A + 16,), jnp.int32),
            pltpu.VMEM((CH_A + 16,), jnp.int32),
            pltpu.VMEM((CH_A,), jnp.int32),
            pltpu.VMEM((CH_A,), jnp.int32),
            pltpu.VMEM((CH_A, 128), jnp.float32),
            pltpu.VMEM((CH_A, 128), jnp.float32),
            pltpu.SemaphoreType.DMA,
            pltpu.SemaphoreType.DMA,
            pltpu.VMEM_SHARED((HK, 128), jnp.float32),
        ],
    )
    def hist_kernel(pk_hbm, out_hbm, ebuf0, ebuf1, lbuf0, lbuf1, ridx0, ridx1,
                    rows0, rows1, sem0, sem1, hist_sh):
        c = lax.axis_index("c")
        s = lax.axis_index("s")
        wid = c * 16 + s
        ebufs = (ebuf0, ebuf1)
        lbufs = (lbuf0, lbuf1)
        ridxs = (ridx0, ridx1)
        rows = (rows0, rows1)
        sems = (sem0, sem1)

        def zinit_body(i, _):
            for j in range(128 // 16):
                rows0[i, pl.ds(j * 16, 16)] = jnp.zeros((16,), jnp.float32)
            return 0
        lax.fori_loop(0, CH_A, zinit_body, 0)
        pltpu.sync_copy(rows0.at[pl.ds(0, HROWS_PER_TILE), :],
                        hist_sh.at[pl.ds(s * HROWS_PER_TILE, HROWS_PER_TILE), :])
        plsc.subcore_barrier()

        iotas = [lax.iota(jnp.int32, 16) + 16 * j for j in range(8)]

        def build_fire(b, g):
            pltpu.sync_copy(pk_hbm.at[wid * NCHUNK_A + g], ebufs[b])
            for j in range(CH_A // 16):
                sl = pl.ds(j * 16, 16)
                kw = ebufs[b][1, sl]
                ridxs[b][sl] = lax.shift_right_logical(kw, 7)
                lbufs[b][sl] = lax.bitwise_and(kw, 127)

            def one_edge(e, _):
                l16 = jnp.full((16,), lbufs[b][pl.ds(e, 16)][0], jnp.int32)
                for j in range(8):
                    rows[b][e, pl.ds(j * 16, 16)] = jnp.where(
                        iotas[j] == l16, 1.0, 0.0)
                return 0
            lax.fori_loop(0, CH_A, one_edge, 0)
            pltpu.async_copy(rows[b], hist_sh.at[ridxs[b]], sems[b], add=True)

        def wait_scatter(b):
            pltpu.make_async_copy(rows[b], hist_sh.at[ridxs[b]],
                                  sems[b]).wait()

        build_fire(0, 0)

        def edge_body(i, _):
            build_fire(1, 2 * i + 1)
            wait_scatter(0)

            @pl.when(2 * i + 2 < NCHUNK_A)
            def _():
                build_fire(0, 2 * i + 2)
            wait_scatter(1)
            return 0
        lax.fori_loop(0, NCHUNK_A // 2, edge_body, 0)
        plsc.subcore_barrier()

        pltpu.sync_copy(hist_sh.at[pl.ds(s * HROWS_PER_TILE, HROWS_PER_TILE), :],
                        out_hbm.at[c, pl.ds(s * HROWS_PER_TILE, HROWS_PER_TILE), :])

    return hist_kernel(pk)


CH_A = 64
NCHUNK_A = EPW // CH_A
EPC = EP // CH_A


def _sc_aggregate(pk, t_tab, w_tab):
    @functools.partial(
        pl.kernel,
        out_type=jax.ShapeDtypeStruct((2, N_ACC, DD), jnp.float32),
        mesh=_mesh,
        compiler_params=pltpu.CompilerParams(needs_layout_passes=False),
        scratch_types=[
            pltpu.VMEM((3, CH_A), jnp.int32),
            pltpu.VMEM((3, CH_A), jnp.int32),
            pltpu.VMEM((CH_A, DD), jnp.float32),
            pltpu.VMEM((CH_A, DD), jnp.float32),
            pltpu.VMEM((CH_A, DD), jnp.float32),
            pltpu.VMEM((CH_A, DD), jnp.float32),
            pltpu.SemaphoreType.DMA,
            pltpu.SemaphoreType.DMA,
            pltpu.VMEM_SHARED((N_ACC, DD), jnp.float32),
        ],
    )
    def agg_kernel(pk_hbm, ttab_hbm, wtab_hbm, out_hbm,
                   ebuf0, ebuf1, rows0, rows1, wrows0, wrows1,
                   sem0, sem1, acc_sh):
        c = lax.axis_index("c")
        s = lax.axis_index("s")
        wid = c * 16 + s
        ebufs = (ebuf0, ebuf1)
        rows = (rows0, rows1)
        wrows = (wrows0, wrows1)
        sems = (sem0, sem1)

        def zinit_body(i, _):
            for j in range(DD // 16):
                rows0[i, pl.ds(j * 16, 16)] = jnp.zeros((16,), jnp.float32)
            return 0
        lax.fori_loop(0, CH_A, zinit_body, 0)

        def zero_body(i, _):
            pltpu.sync_copy(
                rows0, acc_sh.at[pl.ds(s * ROWS_PER_TILE_A + i * CH_A, CH_A), :])
            return 0
        lax.fori_loop(0, 9, zero_body, 0)
        pltpu.sync_copy(rows0.at[pl.ds(0, 56)],
                        acc_sh.at[pl.ds(s * ROWS_PER_TILE_A + 9 * CH_A, 56), :])
        plsc.subcore_barrier()

        def stage_fire(b, g):
            pltpu.sync_copy(pk_hbm.at[wid * NCHUNK_A + g], ebufs[b])
            pltpu.async_copy(ttab_hbm.at[ebufs[b].at[0]], rows[b], sems[b])
            pltpu.async_copy(wtab_hbm.at[ebufs[b].at[1]], wrows[b], sems[b])

        def wait_scale_scatter(b):
            pltpu.make_async_copy(ttab_hbm.at[ebufs[b].at[0]], rows[b],
                                  sems[b]).wait()
            pltpu.make_async_copy(wtab_hbm.at[ebufs[b].at[1]], wrows[b],
                                  sems[b]).wait()

            @plsc.parallel_loop(0, CH_A, unroll=2)
            def scale_body(e):
                for j in range(DD // 16):
                    sl = pl.ds(j * 16, 16)
                    rows[b][e, sl] = rows[b][e, sl] * wrows[b][e, sl]

            pltpu.sync_copy(rows[b], acc_sh.at[ebufs[b].at[2]], add=True)

        stage_fire(0, 0)

        def edge_body(i, _):
            g1 = 2 * i + 1
            g2 = 2 * i + 2
            stage_fire(1, g1)
            wait_scale_scatter(0)

            @pl.when(g2 < NCHUNK_A)
            def _():
                stage_fire(0, g2)
            wait_scale_scatter(1)
            return 0
        lax.fori_loop(0, NCHUNK_A // 2, edge_body, 0)
        plsc.subcore_barrier()

        pltpu.sync_copy(acc_sh.at[pl.ds(s * ROWS_PER_TILE_A, ROWS_PER_TILE_A), :],
                        out_hbm.at[c, pl.ds(s * ROWS_PER_TILE_A, ROWS_PER_TILE_A), :])

    return agg_kernel(pk, t_tab, w_tab)


def _tc_pack_keys(src_p, dst_p, typ_p):
    s2 = src_p.reshape(EPC, CH_A)
    d2 = dst_p.reshape(EPC, CH_A)
    t2 = typ_p.reshape(EPC, CH_A)

    def body(s_ref, d_ref, t_ref, kt_ref, kw_ref, kd_ref):
        t = t_ref[:] * NN
        kt_ref[:] = t + s_ref[:]
        kw_ref[:] = t + d_ref[:]
        kd_ref[:] = d_ref[:]

    kt, kw, kd = pl.pallas_call(
        body,
        grid=(1,),
        in_specs=[pl.BlockSpec((EPC, CH_A), lambda i: (0, 0))] * 3,
        out_specs=[pl.BlockSpec((EPC, CH_A), lambda i: (0, 0))] * 3,
        out_shape=[jax.ShapeDtypeStruct((EPC, CH_A), jnp.int32)] * 3,
    )(s2, d2, t2)
    return jnp.stack([kt, kw, kd], axis=1)


_BN = 400


def _tc_rel_transform(h, wrel):
    def body(h_ref, w_ref, o_ref):
        o_ref[:] = jnp.dot(h_ref[:], w_ref[0],
                           preferred_element_type=jnp.float32)

    nb = NN // _BN
    return pl.pallas_call(
        body,
        grid=(RR, nb),
        in_specs=[
            pl.BlockSpec((_BN, DD), lambda r, n: (n, 0)),
            pl.BlockSpec((1, DD, DD), lambda r, n: (r, 0, 0)),
        ],
        out_specs=pl.BlockSpec((_BN, DD), lambda r, n: (r * nb + n, 0)),
        out_shape=jax.ShapeDtypeStruct((KEYS, DD), jnp.float32),
    )(h, wrel)


def _tc_weight_table(hist2):
    def body(a_ref, b_ref, o_ref):
        cnt = a_ref[0] + b_ref[0]
        o_ref[:] = 1.0 / jnp.maximum(cnt, 1.0)

    return pl.pallas_call(
        body,
        grid=(1,),
        in_specs=[
            pl.BlockSpec((1, HK, 128), lambda i: (0, 0, 0)),
            pl.BlockSpec((1, HK, 128), lambda i: (1, 0, 0)),
        ],
        out_specs=pl.BlockSpec((HK, 128), lambda i: (0, 0)),
        out_shape=jax.ShapeDtypeStruct((HK, 128), jnp.float32),
    )(hist2, hist2)


def _tc_combine(h, wroot, bias, gamma, beta, agg, relu, residual):
    b2 = bias.reshape(1, DD)
    g2 = gamma.reshape(1, DD)
    be2 = beta.reshape(1, DD)

    def body(h_ref, w_ref, b_ref, g_ref, be_ref, a0_ref, a1_ref, o_ref):
        y = jnp.dot(h_ref[:], w_ref[:], preferred_element_type=jnp.float32)
        y = y + b_ref[:] + a0_ref[0] + a1_ref[0]
        mu = jnp.mean(y, axis=-1, keepdims=True)
        yc = y - mu
        var = jnp.mean(yc * yc, axis=-1, keepdims=True)
        y = yc * lax.rsqrt(var + 1e-5) * g_ref[:] + be_ref[:]
        if relu:
            y = jnp.maximum(y, 0.0)
        if residual:
            y = y + h_ref[:]
        o_ref[:] = y

    return pl.pallas_call(
        body,
        grid=(NN // _BN,),
        in_specs=[
            pl.BlockSpec((_BN, DD), lambda n: (n, 0)),
            pl.BlockSpec((DD, DD), lambda n: (0, 0)),
            pl.BlockSpec((1, DD), lambda n: (0, 0)),
            pl.BlockSpec((1, DD), lambda n: (0, 0)),
            pl.BlockSpec((1, DD), lambda n: (0, 0)),
            pl.BlockSpec((1, _BN, DD), lambda n: (0, n, 0)),
            pl.BlockSpec((1, _BN, DD), lambda n: (1, n, 0)),
        ],
        out_specs=pl.BlockSpec((_BN, DD), lambda n: (n, 0)),
        out_shape=jax.ShapeDtypeStruct((NN, DD), jnp.float32),
    )(h, wroot, b2, g2, be2, agg, agg)


def kernel(x, edge_index, edge_type, Wrel0, Wroot0, b0, g0, be0,
           Wrel1, Wroot1, b1, g1, be1):
    src = edge_index[0]
    dst = edge_index[1]
    pad = EP - EE
    src_p = jnp.concatenate([src, jnp.zeros((pad,), jnp.int32)])
    dst_p = jnp.concatenate([dst, jnp.full((pad,), NN, jnp.int32)])
    typ_p = jnp.concatenate([edge_type, jnp.full((pad,), RR - 1, jnp.int32)])

    pk = _tc_pack_keys(src_p, dst_p, typ_p)
    hist2 = _sc_hist(pk)
    w_inv = _tc_weight_table(hist2)
    w_tab = jnp.broadcast_to(w_inv.reshape(HK * 128, 1), (HK * 128, DD))

    t0 = _tc_rel_transform(x, Wrel0)
    agg0 = _sc_aggregate(pk, t0, w_tab)
    h1 = _tc_combine(x, Wroot0, b0, g0, be0, agg0, relu=True, residual=False)

    t1 = _tc_rel_transform(h1, Wrel1)
    agg1 = _sc_aggregate(pk, t1, w_tab)
    h2 = _tc_combine(h1, Wroot1, b1, g1, be1, agg1, relu=False, residual=True)
    return h2


# speedup vs baseline: 1.1972x; 1.0205x over previous
import functools

import jax
import jax.numpy as jnp
from jax import lax
from jax.experimental import pallas as pl
from jax.experimental.pallas import tpu as pltpu
from jax.experimental.pallas import tpu_sc as plsc

NN = 10000
EE = 320000
DD = 128
RR = 10
KEYS = RR * NN
KEYS_PAD = 100096
CHUNK = 128
NTILES = 32
EP = 323584
EPW = EP // NTILES
NCHUNK = EPW // CHUNK
N_ACC = 10112
ROWS_PER_TILE_H = KEYS_PAD // 16
ROWS_PER_TILE_A = N_ACC // 16

_mesh = plsc.VectorSubcoreMesh(core_axis_name="c", subcore_axis_name="s")


HK = 896
HROWS_PER_TILE = HK // 16


def _sc_hist(pk):
    @functools.partial(
        pl.kernel,
        out_type=jax.ShapeDtypeStruct((2, HK, 128), jnp.float32),
        mesh=_mesh,
        compiler_params=pltpu.CompilerParams(needs_layout_passes=False),
        scratch_types=[
            pltpu.VMEM((3, CH_A), jnp.int32),
            pltpu.VMEM((3, CH_A), jnp.int32),
            pltpu.VMEM((CH_A + 16,), jnp.int32),
            pltpu.VMEM((CH_A + 16,), jnp.int32),
            pltpu.VMEM((CH_A,), jnp.int32),
            pltpu.VMEM((CH_A,), jnp.int32),
            pltpu.VMEM((CH_A, 128), jnp.float32),
            pltpu.VMEM((CH_A, 128), jnp.float32),
            pltpu.SemaphoreType.DMA,
            pltpu.SemaphoreType.DMA,
            pltpu.VMEM_SHARED((HK, 128), jnp.float32),
        ],
    )
    def hist_kernel(pk_hbm, out_hbm, ebuf0, ebuf1, lbuf0, lbuf1, ridx0, ridx1,
                    rows0, rows1, sem0, sem1, hist_sh):
        c = lax.axis_index("c")
        s = lax.axis_index("s")
        wid = c * 16 + s
        ebufs = (ebuf0, ebuf1)
        lbufs = (lbuf0, lbuf1)
        ridxs = (ridx0, ridx1)
        rows = (rows0, rows1)
        sems = (sem0, sem1)

        def zinit_body(i, _):
            for j in range(128 // 16):
                rows0[i, pl.ds(j * 16, 16)] = jnp.zeros((16,), jnp.float32)
            return 0
        lax.fori_loop(0, CH_A, zinit_body, 0)
        pltpu.sync_copy(rows0.at[pl.ds(0, HROWS_PER_TILE), :],
                        hist_sh.at[pl.ds(s * HROWS_PER_TILE, HROWS_PER_TILE), :])
        plsc.subcore_barrier()

        iotas = [lax.iota(jnp.int32, 16) + 16 * j for j in range(8)]

        def build_fire(b, g):
            pltpu.sync_copy(pk_hbm.at[wid * NCHUNK_A + g], ebufs[b])
            for j in range(CH_A // 16):
                sl = pl.ds(j * 16, 16)
                kw = ebufs[b][1, sl]
                ridxs[b][sl] = lax.shift_right_logical(kw, 7)
                lbufs[b][sl] = lax.bitwise_and(kw, 127)

            @plsc.parallel_loop(0, CH_A, unroll=2)
            def one_edge(e):
                l16 = jnp.full((16,), lbufs[b][pl.ds(e, 16)][0], jnp.int32)
                for j in range(8):
                    rows[b][e, pl.ds(j * 16, 16)] = jnp.where(
                        iotas[j] == l16, 1.0, 0.0)
            pltpu.async_copy(rows[b], hist_sh.at[ridxs[b]], sems[b], add=True)

        def wait_scatter(b):
            pltpu.make_async_copy(rows[b], hist_sh.at[ridxs[b]],
                                  sems[b]).wait()

        build_fire(0, 0)

        def edge_body(i, _):
            build_fire(1, 2 * i + 1)
            wait_scatter(0)

            @pl.when(2 * i + 2 < NCHUNK_A)
            def _():
                build_fire(0, 2 * i + 2)
            wait_scatter(1)
            return 0
        lax.fori_loop(0, NCHUNK_A // 2, edge_body, 0)
        plsc.subcore_barrier()

        pltpu.sync_copy(hist_sh.at[pl.ds(s * HROWS_PER_TILE, HROWS_PER_TILE), :],
                        out_hbm.at[c, pl.ds(s * HROWS_PER_TILE, HROWS_PER_TILE), :])

    return hist_kernel(pk)


CH_A = 64
NCHUNK_A = EPW // CH_A
EPC = EP // CH_A


def _sc_aggregate(pk, t_tab, w_tab):
    @functools.partial(
        pl.kernel,
        out_type=jax.ShapeDtypeStruct((2, N_ACC, DD), jnp.float32),
        mesh=_mesh,
        compiler_params=pltpu.CompilerParams(needs_layout_passes=False),
        scratch_types=[
            pltpu.VMEM((3, CH_A), jnp.int32),
            pltpu.VMEM((3, CH_A), jnp.int32),
            pltpu.VMEM((CH_A, DD), jnp.float32),
            pltpu.VMEM((CH_A, DD), jnp.float32),
            pltpu.VMEM((CH_A, DD), jnp.float32),
            pltpu.VMEM((CH_A, DD), jnp.float32),
            pltpu.SemaphoreType.DMA,
            pltpu.SemaphoreType.DMA,
            pltpu.VMEM_SHARED((N_ACC, DD), jnp.float32),
        ],
    )
    def agg_kernel(pk_hbm, ttab_hbm, wtab_hbm, out_hbm,
                   ebuf0, ebuf1, rows0, rows1, wrows0, wrows1,
                   sem0, sem1, acc_sh):
        c = lax.axis_index("c")
        s = lax.axis_index("s")
        wid = c * 16 + s
        ebufs = (ebuf0, ebuf1)
        rows = (rows0, rows1)
        wrows = (wrows0, wrows1)
        sems = (sem0, sem1)

        def zinit_body(i, _):
            for j in range(DD // 16):
                rows0[i, pl.ds(j * 16, 16)] = jnp.zeros((16,), jnp.float32)
            return 0
        lax.fori_loop(0, CH_A, zinit_body, 0)

        def zero_body(i, _):
            pltpu.sync_copy(
                rows0, acc_sh.at[pl.ds(s * ROWS_PER_TILE_A + i * CH_A, CH_A), :])
            return 0
        lax.fori_loop(0, 9, zero_body, 0)
        pltpu.sync_copy(rows0.at[pl.ds(0, 56)],
                        acc_sh.at[pl.ds(s * ROWS_PER_TILE_A + 9 * CH_A, 56), :])
        plsc.subcore_barrier()

        def stage_fire(b, g):
            pltpu.sync_copy(pk_hbm.at[wid * NCHUNK_A + g], ebufs[b])
            pltpu.async_copy(ttab_hbm.at[ebufs[b].at[0]], rows[b], sems[b])
            pltpu.async_copy(wtab_hbm.at[ebufs[b].at[1]], wrows[b], sems[b])

        def wait_scale_scatter(b):
            pltpu.make_async_copy(ttab_hbm.at[ebufs[b].at[0]], rows[b],
                                  sems[b]).wait()
            pltpu.make_async_copy(wtab_hbm.at[ebufs[b].at[1]], wrows[b],
                                  sems[b]).wait()

            @plsc.parallel_loop(0, CH_A, unroll=4)
            def scale_body(e):
                for j in range(DD // 16):
                    sl = pl.ds(j * 16, 16)
                    rows[b][e, sl] = rows[b][e, sl] * wrows[b][e, sl]

            pltpu.sync_copy(rows[b], acc_sh.at[ebufs[b].at[2]], add=True)

        stage_fire(0, 0)

        def edge_body(i, _):
            g1 = 2 * i + 1
            g2 = 2 * i + 2
            stage_fire(1, g1)
            wait_scale_scatter(0)

            @pl.when(g2 < NCHUNK_A)
            def _():
                stage_fire(0, g2)
            wait_scale_scatter(1)
            return 0
        lax.fori_loop(0, NCHUNK_A // 2, edge_body, 0)
        plsc.subcore_barrier()

        pltpu.sync_copy(acc_sh.at[pl.ds(s * ROWS_PER_TILE_A, ROWS_PER_TILE_A), :],
                        out_hbm.at[c, pl.ds(s * ROWS_PER_TILE_A, ROWS_PER_TILE_A), :])

    return agg_kernel(pk, t_tab, w_tab)


def _tc_pack_keys(src_p, dst_p, typ_p):
    s2 = src_p.reshape(EPC, CH_A)
    d2 = dst_p.reshape(EPC, CH_A)
    t2 = typ_p.reshape(EPC, CH_A)

    def body(s_ref, d_ref, t_ref, kt_ref, kw_ref, kd_ref):
        t = t_ref[:] * NN
        kt_ref[:] = t + s_ref[:]
        kw_ref[:] = t + d_ref[:]
        kd_ref[:] = d_ref[:]

    kt, kw, kd = pl.pallas_call(
        body,
        grid=(1,),
        in_specs=[pl.BlockSpec((EPC, CH_A), lambda i: (0, 0))] * 3,
        out_specs=[pl.BlockSpec((EPC, CH_A), lambda i: (0, 0))] * 3,
        out_shape=[jax.ShapeDtypeStruct((EPC, CH_A), jnp.int32)] * 3,
    )(s2, d2, t2)
    return jnp.stack([kt, kw, kd], axis=1)


_BN = 400


def _tc_rel_transform(h, wrel):
    def body(h_ref, w_ref, o_ref):
        o_ref[:] = jnp.dot(h_ref[:], w_ref[0],
                           preferred_element_type=jnp.float32)

    nb = NN // _BN
    return pl.pallas_call(
        body,
        grid=(RR, nb),
        in_specs=[
            pl.BlockSpec((_BN, DD), lambda r, n: (n, 0)),
            pl.BlockSpec((1, DD, DD), lambda r, n: (r, 0, 0)),
        ],
        out_specs=pl.BlockSpec((_BN, DD), lambda r, n: (r * nb + n, 0)),
        out_shape=jax.ShapeDtypeStruct((KEYS, DD), jnp.float32),
    )(h, wrel)


def _tc_weight_table(hist2):
    def body(a_ref, b_ref, o_ref):
        cnt = a_ref[0] + b_ref[0]
        o_ref[:] = 1.0 / jnp.maximum(cnt, 1.0)

    return pl.pallas_call(
        body,
        grid=(1,),
        in_specs=[
            pl.BlockSpec((1, HK, 128), lambda i: (0, 0, 0)),
            pl.BlockSpec((1, HK, 128), lambda i: (1, 0, 0)),
        ],
        out_specs=pl.BlockSpec((HK, 128), lambda i: (0, 0)),
        out_shape=jax.ShapeDtypeStruct((HK, 128), jnp.float32),
    )(hist2, hist2)


def _tc_combine(h, wroot, bias, gamma, beta, agg, relu, residual):
    b2 = bias.reshape(1, DD)
    g2 = gamma.reshape(1, DD)
    be2 = beta.reshape(1, DD)

    def body(h_ref, w_ref, b_ref, g_ref, be_ref, a0_ref, a1_ref, o_ref):
        y = jnp.dot(h_ref[:], w_ref[:], preferred_element_type=jnp.float32)
        y = y + b_ref[:] + a0_ref[0] + a1_ref[0]
        mu = jnp.mean(y, axis=-1, keepdims=True)
        yc = y - mu
        var = jnp.mean(yc * yc, axis=-1, keepdims=True)
        y = yc * lax.rsqrt(var + 1e-5) * g_ref[:] + be_ref[:]
        if relu:
            y = jnp.maximum(y, 0.0)
        if residual:
            y = y + h_ref[:]
        o_ref[:] = y

    return pl.pallas_call(
        body,
        grid=(NN // _BN,),
        in_specs=[
            pl.BlockSpec((_BN, DD), lambda n: (n, 0)),
            pl.BlockSpec((DD, DD), lambda n: (0, 0)),
            pl.BlockSpec((1, DD), lambda n: (0, 0)),
            pl.BlockSpec((1, DD), lambda n: (0, 0)),
            pl.BlockSpec((1, DD), lambda n: (0, 0)),
            pl.BlockSpec((1, _BN, DD), lambda n: (0, n, 0)),
            pl.BlockSpec((1, _BN, DD), lambda n: (1, n, 0)),
        ],
        out_specs=pl.BlockSpec((_BN, DD), lambda n: (n, 0)),
        out_shape=jax.ShapeDtypeStruct((NN, DD), jnp.float32),
    )(h, wroot, b2, g2, be2, agg, agg)


def kernel(x, edge_index, edge_type, Wrel0, Wroot0, b0, g0, be0,
           Wrel1, Wroot1, b1, g1, be1):
    src = edge_index[0]
    dst = edge_index[1]
    pad = EP - EE
    src_p = jnp.concatenate([src, jnp.zeros((pad,), jnp.int32)])
    dst_p = jnp.concatenate([dst, jnp.full((pad,), NN, jnp.int32)])
    typ_p = jnp.concatenate([edge_type, jnp.full((pad,), RR - 1, jnp.int32)])

    pk = _tc_pack_keys(src_p, dst_p, typ_p)
    hist2 = _sc_hist(pk)
    w_inv = _tc_weight_table(hist2)
    w_tab = jnp.broadcast_to(w_inv.reshape(HK * 128, 1), (HK * 128, DD))

    t0 = _tc_rel_transform(x, Wrel0)
    agg0 = _sc_aggregate(pk, t0, w_tab)
    h1 = _tc_combine(x, Wroot0, b0, g0, be0, agg0, relu=True, residual=False)

    t1 = _tc_rel_transform(h1, Wrel1)
    agg1 = _sc_aggregate(pk, t1, w_tab)
    h2 = _tc_combine(h1, Wroot1, b1, g1, be1, agg1, relu=False, residual=True)
    return h2
